# baseline (speedup 1.0000x reference)
import functools

import jax
import jax.numpy as jnp
from jax import lax
from jax.experimental import pallas as pl
from jax.experimental.pallas import tpu as pltpu
from jax.experimental.pallas import tpu_sc as plsc

N = 10000
D_IN = 128
D_HID = 64
D_OUT = 16
N_CLASS = 10

NC = 1
NS = 16
L = 16
NW = NC * NS

NPAD = 10240
NPT = NPAD // NS

E = 320000
CH = 128
NG = CH // L
P2C = 158
PCA = P2C + 2
EWK = P2C * CH
EPAD = EWK * NW

RB = 2000
RBP = NPAD // 5

_mesh = plsc.VectorSubcoreMesh(core_axis_name="c", subcore_axis_name="s",
                               num_cores=NC, num_subcores=NS)


def _zero_rows(rows, ncols):
    z = jnp.zeros((L,), jnp.float32)

    def body(i, _):
        for d in range(ncols // L):
            rows[i, pl.ds(d * L, L)] = z
        return 0

    lax.fori_loop(0, CH, body, 0)


def _deg_body(col_hbm, ew_hbm, deg_hbm, colb, ewb, zb, deg_sh):
    cid = lax.axis_index("c")
    sid = lax.axis_index("s")
    t0 = sid * NPT
    z = jnp.zeros((L,), jnp.float32)

    def zdeg(v, _):
        zb[pl.ds(v * L, L)] = z
        return 0

    lax.fori_loop(0, NPT // L, zdeg, 0)
    pltpu.sync_copy(zb, deg_sh.at[pl.ds(t0, NPT)])

    wid = cid * NS + sid
    wb = wid * PCA
    pltpu.sync_copy(col_hbm.at[pl.ds(wb, P2C)], colb)
    pltpu.sync_copy(ew_hbm.at[pl.ds(wb, P2C)], ewb)
    plsc.subcore_barrier()

    def p1(c, _):
        pltpu.sync_copy(ewb.at[c], deg_sh.at[colb.at[c]], add=True)
        return 0

    lax.fori_loop(0, P2C, p1, 0)
    plsc.subcore_barrier()
    pltpu.sync_copy(deg_sh.at[pl.ds(t0, NPT)], deg_hbm.at[cid, pl.ds(t0, NPT)])


_sc_deg = functools.partial(
    pl.kernel,
    out_type=jax.ShapeDtypeStruct((NC, NPAD), jnp.float32),
    mesh=_mesh,
    scratch_types=[
        pltpu.VMEM((P2C, CH), jnp.int32),
        pltpu.VMEM((P2C, CH), jnp.float32),
        pltpu.VMEM((NPT,), jnp.float32),
        pltpu.VMEM_SHARED((NPAD,), jnp.float32),
    ],
    compiler_params=pltpu.CompilerParams(needs_layout_passes=False, use_tc_tiling_on_sc=False),
)(_deg_body)


def _make_agg(ncols, npasses=1):
    def body(row_hbm, col_hbm, ew_hbm, g_hbm, agg_hbm,
             rowb, colb, ewb, g0, g1, s0, s1, g_sh, acc_sh,
             gsem0, gsem1, ssem0, ssem1):
        cid = lax.axis_index("c")
        sid = lax.axis_index("s")
        t0 = sid * NPT

        wid = cid * NS + sid
        wb = wid * PCA
        pltpu.sync_copy(row_hbm.at[pl.ds(wb, PCA)], rowb)
        pltpu.sync_copy(col_hbm.at[pl.ds(wb, P2C)], colb)
        pltpu.sync_copy(ew_hbm.at[pl.ds(wb, P2C)], ewb)

        gbufs = (g0, g1)
        gsems = (gsem0, gsem1)
        sbufs = (s0, s1)
        ssems = (ssem0, ssem1)

        def scale(c, gb, sb):
            def grp(g, _):
                wv = ewb[c, pl.ds(g * L, L)]
                for j in range(L):
                    w_s = wv[j]
                    r = g * L + j
                    for d in range(ncols // L):
                        sb[r, pl.ds(d * L, L)] = gb[r, pl.ds(d * L, L)] * w_s
                return 0

            lax.fori_loop(0, NG, grp, 0)

        for p in range(npasses):
            co = p * ncols
            r0 = sid * (N // NS)
            pltpu.sync_copy(g_hbm.at[pl.ds(r0, N // NS), pl.ds(co, ncols)],
                            g_sh.at[pl.ds(r0, N // NS)])

            _zero_rows(s0, ncols)
            for k in range(NPT // CH):
                pltpu.sync_copy(s0, acc_sh.at[pl.ds(t0 + k * CH, CH)])
            plsc.subcore_barrier()

            for b in range(2):
                pltpu.async_copy(g_sh.at[rowb.at[b]], gbufs[b], gsems[b])

            for b in range(2):
                gb, sb = gbufs[b], sbufs[b]
                pltpu.make_async_copy(g_sh.at[rowb.at[b]], gb,
                                      gsems[b]).wait()
                scale(b, gb, sb)
                pltpu.async_copy(g_sh.at[rowb.at[b + 2]], gb, gsems[b])
                pltpu.async_copy(sb, acc_sh.at[colb.at[b]], ssems[b],
                                 add=True)

            def p2(o, _):
                for b in range(2):
                    c = o * 2 + b
                    gb, sb = gbufs[b], sbufs[b]
                    pltpu.make_async_copy(g_sh.at[rowb.at[c]], gb,
                                          gsems[b]).wait()
                    pltpu.make_async_copy(sb, acc_sh.at[colb.at[c - 2]],
                                          ssems[b]).wait()
                    scale(c, gb, sb)
                    pltpu.async_copy(g_sh.at[rowb.at[c + 2]], gb, gsems[b])
                    pltpu.async_copy(sb, acc_sh.at[colb.at[c]], ssems[b],
                                     add=True)
                return 0

            lax.fori_loop(1, P2C // 2, p2, 0)
            for b in range(2):
                pltpu.make_async_copy(g_sh.at[rowb.at[P2C + b]], gbufs[b],
                                      gsems[b]).wait()
                pltpu.make_async_copy(sbufs[b],
                                      acc_sh.at[colb.at[P2C - 2 + b]],
                                      ssems[b]).wait()
            plsc.subcore_barrier()
            pltpu.sync_copy(acc_sh.at[pl.ds(t0, NPT)],
                            agg_hbm.at[cid, pl.ds(t0, NPT),
                                       pl.ds(co, ncols)])

    return functools.partial(
        pl.kernel,
        out_type=jax.ShapeDtypeStruct((NC, NPAD, ncols * npasses),
                                      jnp.float32),
        mesh=_mesh,
        scratch_types=[
            pltpu.VMEM((PCA, CH), jnp.int32),
            pltpu.VMEM((P2C, CH), jnp.int32),
            pltpu.VMEM((P2C, CH), jnp.float32),
            pltpu.VMEM((CH, ncols), jnp.float32),
            pltpu.VMEM((CH, ncols), jnp.float32),
            pltpu.VMEM((CH, ncols), jnp.float32),
            pltpu.VMEM((CH, ncols), jnp.float32),
            pltpu.VMEM_SHARED((N, ncols), jnp.float32),
            pltpu.VMEM_SHARED((NPAD, ncols), jnp.float32),
            pltpu.SemaphoreType.DMA,
            pltpu.SemaphoreType.DMA,
            pltpu.SemaphoreType.DMA,
            pltpu.SemaphoreType.DMA,
        ],
        compiler_params=pltpu.CompilerParams(needs_layout_passes=False, use_tc_tiling_on_sc=False),
    )(body)


_sc_agg1 = _make_agg(D_HID // 2, npasses=2)
_sc_agg2 = _make_agg(D_OUT)


def _lin1_body(x_ref, w_ref, da0_ref, db0_ref, g_ref, dis_ref):
    h = jnp.dot(x_ref[...], w_ref[...], preferred_element_type=jnp.float32)
    one = jnp.float32(1.0)
    g_ref[...] = lax.rsqrt(da0_ref[...] + one) * h
    dis_ref[...] = lax.rsqrt(db0_ref[...] + one)


def _mid_body(a0_ref, g1_ref, dis_ref, b1_ref, w2_ref, g2_ref):
    d = dis_ref[...]
    o1 = d * (a0_ref[...] + g1_ref[...]) + b1_ref[...]
    h = jnp.maximum(o1, jnp.float32(0.0))
    g2_ref[...] = d * jnp.dot(h, w2_ref[...],
                              preferred_element_type=jnp.float32)


def _out_body(a0_ref, g2_ref, dis_ref, b2_ref, o_ref):
    d = dis_ref[...]
    o2 = d * (a0_ref[...] + g2_ref[...]) + b2_ref[...]
    colid = lax.broadcasted_iota(jnp.int32, o2.shape, 1)
    valid = colid < N_CLASS
    m = jnp.max(jnp.where(valid, o2, jnp.float32(-1e30)), axis=1,
                keepdims=True)
    ex = jnp.where(valid, jnp.exp(o2 - m), jnp.float32(0.0))
    s = jnp.sum(ex, axis=1, keepdims=True)
    o_ref[...] = (o2 - m) - jnp.log(s)


def _row_spec(ncols):
    return pl.BlockSpec((RB, ncols), lambda i: (i, 0))


def _full_spec(shape):
    return pl.BlockSpec(shape, lambda i: tuple(0 for _ in shape))


def kernel(x, edge_index, edge_weight, W1, b1, W2, b2):
    row = edge_index[0].astype(jnp.int32)
    col = edge_index[1].astype(jnp.int32)
    ew = edge_weight.astype(jnp.float32)
    padn = EPAD - row.shape[0]

    def _chunked(a):
        a = jnp.concatenate([a, jnp.zeros((padn,), a.dtype)])
        a = a.reshape(NW, P2C, CH)
        a = jnp.pad(a, ((0, 0), (0, PCA - P2C), (0, 0)))
        return a.reshape(NW * PCA, CH)

    rowp = _chunked(row)
    colp = _chunked(col)
    ewp = _chunked(ew)
    W2p = jnp.pad(W2, ((0, 0), (0, D_OUT - N_CLASS)))
    b2p = jnp.pad(b2, (0, D_OUT - N_CLASS))

    degp = _sc_deg(colp, ewp)
    d0 = degp[0, :, None]

    g1, dis2d = pl.pallas_call(
        _lin1_body,
        grid=(N // RB,),
        in_specs=[_row_spec(D_IN), _full_spec((D_IN, D_HID)),
                  pl.BlockSpec((RB, 1), lambda i: (i, 0)),
                  pl.BlockSpec((RBP, 1), lambda i: (i, 0))],
        out_specs=[_row_spec(D_HID), pl.BlockSpec((RBP, 1), lambda i: (i, 0))],
        out_shape=[jax.ShapeDtypeStruct((N, D_HID), jnp.float32),
                   jax.ShapeDtypeStruct((NPAD, 1), jnp.float32)],
    )(x, W1, d0[:N], d0)

    agg1 = _sc_agg1(rowp, colp, ewp, g1)
    disc = dis2d[:N]

    g2 = pl.pallas_call(
        _mid_body,
        grid=(N // RB,),
        in_specs=[_row_spec(D_HID), _row_spec(D_HID),
                  pl.BlockSpec((RB, 1), lambda i: (i, 0)),
                  _full_spec((1, D_HID)), _full_spec((D_HID, D_OUT))],
        out_specs=_row_spec(D_OUT),
        out_shape=jax.ShapeDtypeStruct((N, D_OUT), jnp.float32),
    )(agg1[0, :N], g1, disc, b1[None, :], W2p)

    agg2 = _sc_agg2(rowp, colp, ewp, g2)

    o = pl.pallas_call(
        _out_body,
        grid=(N // RB,),
        in_specs=[_row_spec(D_OUT), _row_spec(D_OUT),
                  pl.BlockSpec((RB, 1), lambda i: (i, 0)),
                  _full_spec((1, D_OUT))],
        out_specs=_row_spec(D_OUT),
        out_shape=jax.ShapeDtypeStruct((N, D_OUT), jnp.float32),
    )(agg2[0, :N], g2, disc, b2p[None, :])

    return o[:, :N_CLASS]


# speedup vs baseline: 1.1565x; 1.1565x over previous
import functools

import jax
import jax.numpy as jnp
from jax import lax
from jax.experimental import pallas as pl
from jax.experimental.pallas import tpu as pltpu
from jax.experimental.pallas import tpu_sc as plsc

N = 10000
D_IN = 128
D_HID = 64
D_OUT = 16
N_CLASS = 10

NC = 2
NS = 16
L = 16
NW = NC * NS

NPAD = 10240
NPT = NPAD // NS

E = 320000
CH = 128
NG = CH // L
P2C = 80
PCA = P2C + 2
EWK = P2C * CH
EPAD = EWK * NW

RB = 2000
RBP = NPAD // 5

_mesh = plsc.VectorSubcoreMesh(core_axis_name="c", subcore_axis_name="s",
                               num_cores=NC, num_subcores=NS)


def _zero_rows(rows, ncols):
    z = jnp.zeros((L,), jnp.float32)

    def body(i, _):
        for d in range(ncols // L):
            rows[i, pl.ds(d * L, L)] = z
        return 0

    lax.fori_loop(0, CH, body, 0)


def _deg_body(col_hbm, ew_hbm, deg_hbm, colb, ewb, zb, deg_sh):
    cid = lax.axis_index("c")
    sid = lax.axis_index("s")
    t0 = sid * NPT
    z = jnp.zeros((L,), jnp.float32)

    def zdeg(v, _):
        zb[pl.ds(v * L, L)] = z
        return 0

    lax.fori_loop(0, NPT // L, zdeg, 0)
    pltpu.sync_copy(zb, deg_sh.at[pl.ds(t0, NPT)])

    wid = cid * NS + sid
    wb = wid * PCA
    pltpu.sync_copy(col_hbm.at[pl.ds(wb, P2C)], colb)
    pltpu.sync_copy(ew_hbm.at[pl.ds(wb, P2C)], ewb)
    plsc.subcore_barrier()

    def p1(c, _):
        pltpu.sync_copy(ewb.at[c], deg_sh.at[colb.at[c]], add=True)
        return 0

    lax.fori_loop(0, P2C, p1, 0)
    plsc.subcore_barrier()
    pltpu.sync_copy(deg_sh.at[pl.ds(t0, NPT)], deg_hbm.at[cid, pl.ds(t0, NPT)])


_sc_deg = functools.partial(
    pl.kernel,
    out_type=jax.ShapeDtypeStruct((NC, NPAD), jnp.float32),
    mesh=_mesh,
    scratch_types=[
        pltpu.VMEM((P2C, CH), jnp.int32),
        pltpu.VMEM((P2C, CH), jnp.float32),
        pltpu.VMEM((NPT,), jnp.float32),
        pltpu.VMEM_SHARED((NPAD,), jnp.float32),
    ],
    compiler_params=pltpu.CompilerParams(needs_layout_passes=False, use_tc_tiling_on_sc=False),
)(_deg_body)


def _make_agg(ncols, npasses=1):
    def body(row_hbm, col_hbm, ew_hbm, g_hbm, agg_hbm,
             rowb, colb, ewb, g0, g1, s0, s1, g_sh, acc_sh,
             gsem0, gsem1, ssem0, ssem1):
        cid = lax.axis_index("c")
        sid = lax.axis_index("s")
        t0 = sid * NPT

        wid = cid * NS + sid
        wb = wid * PCA
        pltpu.sync_copy(row_hbm.at[pl.ds(wb, PCA)], rowb)
        pltpu.sync_copy(col_hbm.at[pl.ds(wb, P2C)], colb)
        pltpu.sync_copy(ew_hbm.at[pl.ds(wb, P2C)], ewb)

        gbufs = (g0, g1)
        gsems = (gsem0, gsem1)
        sbufs = (s0, s1)
        ssems = (ssem0, ssem1)

        def scale(c, gb, sb):
            def grp(g, _):
                wv = ewb[c, pl.ds(g * L, L)]
                for j in range(L):
                    w_s = wv[j]
                    r = g * L + j
                    for d in range(ncols // L):
                        sb[r, pl.ds(d * L, L)] = gb[r, pl.ds(d * L, L)] * w_s
                return 0

            lax.fori_loop(0, NG, grp, 0)

        for p in range(npasses):
            co = p * ncols
            r0 = sid * (N // NS)
            pltpu.sync_copy(g_hbm.at[pl.ds(r0, N // NS), pl.ds(co, ncols)],
                            g_sh.at[pl.ds(r0, N // NS)])

            _zero_rows(s0, ncols)
            for k in range(NPT // CH):
                pltpu.sync_copy(s0, acc_sh.at[pl.ds(t0 + k * CH, CH)])
            plsc.subcore_barrier()

            for b in range(2):
                pltpu.async_copy(g_sh.at[rowb.at[b]], gbufs[b], gsems[b])

            for b in range(2):
                gb, sb = gbufs[b], sbufs[b]
                pltpu.make_async_copy(g_sh.at[rowb.at[b]], gb,
                                      gsems[b]).wait()
                scale(b, gb, sb)
                pltpu.async_copy(g_sh.at[rowb.at[b + 2]], gb, gsems[b])
                pltpu.async_copy(sb, acc_sh.at[colb.at[b]], ssems[b],
                                 add=True)

            def p2(o, _):
                for b in range(2):
                    c = o * 2 + b
                    gb, sb = gbufs[b], sbufs[b]
                    pltpu.make_async_copy(g_sh.at[rowb.at[c]], gb,
                                          gsems[b]).wait()
                    pltpu.make_async_copy(sb, acc_sh.at[colb.at[c - 2]],
                                          ssems[b]).wait()
                    scale(c, gb, sb)
                    pltpu.async_copy(g_sh.at[rowb.at[c + 2]], gb, gsems[b])
                    pltpu.async_copy(sb, acc_sh.at[colb.at[c]], ssems[b],
                                     add=True)
                return 0

            lax.fori_loop(1, P2C // 2, p2, 0)
            for b in range(2):
                pltpu.make_async_copy(g_sh.at[rowb.at[P2C + b]], gbufs[b],
                                      gsems[b]).wait()
                pltpu.make_async_copy(sbufs[b],
                                      acc_sh.at[colb.at[P2C - 2 + b]],
                                      ssems[b]).wait()
            plsc.subcore_barrier()
            pltpu.sync_copy(acc_sh.at[pl.ds(t0, NPT)],
                            agg_hbm.at[cid, pl.ds(t0, NPT),
                                       pl.ds(co, ncols)])

    return functools.partial(
        pl.kernel,
        out_type=jax.ShapeDtypeStruct((NC, NPAD, ncols * npasses),
                                      jnp.float32),
        mesh=_mesh,
        scratch_types=[
            pltpu.VMEM((PCA, CH), jnp.int32),
            pltpu.VMEM((P2C, CH), jnp.int32),
            pltpu.VMEM((P2C, CH), jnp.float32),
            pltpu.VMEM((CH, ncols), jnp.float32),
            pltpu.VMEM((CH, ncols), jnp.float32),
            pltpu.VMEM((CH, ncols), jnp.float32),
            pltpu.VMEM((CH, ncols), jnp.float32),
            pltpu.VMEM_SHARED((N, ncols), jnp.float32),
            pltpu.VMEM_SHARED((NPAD, ncols), jnp.float32),
            pltpu.SemaphoreType.DMA,
            pltpu.SemaphoreType.DMA,
            pltpu.SemaphoreType.DMA,
            pltpu.SemaphoreType.DMA,
        ],
        compiler_params=pltpu.CompilerParams(needs_layout_passes=False, use_tc_tiling_on_sc=False),
    )(body)


_sc_agg1 = _make_agg(D_HID // 2, npasses=2)
_sc_agg2 = _make_agg(D_OUT)


def _lin1_body(x_ref, w_ref, da0_ref, da1_ref, db0_ref, db1_ref,
               g_ref, dis_ref):
    h = jnp.dot(x_ref[...], w_ref[...], preferred_element_type=jnp.float32)
    one = jnp.float32(1.0)
    g_ref[...] = lax.rsqrt(da0_ref[...] + da1_ref[...] + one) * h
    dis_ref[...] = lax.rsqrt(db0_ref[...] + db1_ref[...] + one)


def _mid_body(a0_ref, a1_ref, g1_ref, dis_ref, b1_ref, w2_ref, g2_ref):
    d = dis_ref[...]
    o1 = d * (a0_ref[...] + a1_ref[...] + g1_ref[...]) + b1_ref[...]
    h = jnp.maximum(o1, jnp.float32(0.0))
    g2_ref[...] = d * jnp.dot(h, w2_ref[...],
                              preferred_element_type=jnp.float32)


def _out_body(a0_ref, a1_ref, g2_ref, dis_ref, b2_ref, o_ref):
    d = dis_ref[...]
    o2 = d * (a0_ref[...] + a1_ref[...] + g2_ref[...]) + b2_ref[...]
    colid = lax.broadcasted_iota(jnp.int32, o2.shape, 1)
    valid = colid < N_CLASS
    m = jnp.max(jnp.where(valid, o2, jnp.float32(-1e30)), axis=1,
                keepdims=True)
    ex = jnp.where(valid, jnp.exp(o2 - m), jnp.float32(0.0))
    s = jnp.sum(ex, axis=1, keepdims=True)
    o_ref[...] = (o2 - m) - jnp.log(s)


def _row_spec(ncols):
    return pl.BlockSpec((RB, ncols), lambda i: (i, 0))


def _full_spec(shape):
    return pl.BlockSpec(shape, lambda i: tuple(0 for _ in shape))


def kernel(x, edge_index, edge_weight, W1, b1, W2, b2):
    row = edge_index[0].astype(jnp.int32)
    col = edge_index[1].astype(jnp.int32)
    ew = edge_weight.astype(jnp.float32)
    padn = EPAD - row.shape[0]

    def _chunked(a):
        a = jnp.concatenate([a, jnp.zeros((padn,), a.dtype)])
        a = a.reshape(NW, P2C, CH)
        a = jnp.pad(a, ((0, 0), (0, PCA - P2C), (0, 0)))
        return a.reshape(NW * PCA, CH)

    rowp = _chunked(row)
    colp = _chunked(col)
    ewp = _chunked(ew)
    W2p = jnp.pad(W2, ((0, 0), (0, D_OUT - N_CLASS)))
    b2p = jnp.pad(b2, (0, D_OUT - N_CLASS))

    degp = _sc_deg(colp, ewp)
    d0 = degp[0, :, None]
    d1 = degp[1, :, None]

    g1, dis2d = pl.pallas_call(
        _lin1_body,
        grid=(N // RB,),
        in_specs=[_row_spec(D_IN), _full_spec((D_IN, D_HID)),
                  pl.BlockSpec((RB, 1), lambda i: (i, 0)),
                  pl.BlockSpec((RB, 1), lambda i: (i, 0)),
                  pl.BlockSpec((RBP, 1), lambda i: (i, 0)),
                  pl.BlockSpec((RBP, 1), lambda i: (i, 0))],
        out_specs=[_row_spec(D_HID), pl.BlockSpec((RBP, 1), lambda i: (i, 0))],
        out_shape=[jax.ShapeDtypeStruct((N, D_HID), jnp.float32),
                   jax.ShapeDtypeStruct((NPAD, 1), jnp.float32)],
    )(x, W1, d0[:N], d1[:N], d0, d1)

    agg1 = _sc_agg1(rowp, colp, ewp, g1)
    disc = dis2d[:N]

    g2 = pl.pallas_call(
        _mid_body,
        grid=(N // RB,),
        in_specs=[_row_spec(D_HID), _row_spec(D_HID), _row_spec(D_HID),
                  pl.BlockSpec((RB, 1), lambda i: (i, 0)),
                  _full_spec((1, D_HID)), _full_spec((D_HID, D_OUT))],
        out_specs=_row_spec(D_OUT),
        out_shape=jax.ShapeDtypeStruct((N, D_OUT), jnp.float32),
    )(agg1[0, :N], agg1[1, :N], g1, disc, b1[None, :], W2p)

    agg2 = _sc_agg2(rowp, colp, ewp, g2)

    o = pl.pallas_call(
        _out_body,
        grid=(N // RB,),
        in_specs=[_row_spec(D_OUT), _row_spec(D_OUT), _row_spec(D_OUT),
                  pl.BlockSpec((RB, 1), lambda i: (i, 0)),
                  _full_spec((1, D_OUT))],
        out_specs=_row_spec(D_OUT),
        out_shape=jax.ShapeDtypeStruct((N, D_OUT), jnp.float32),
    )(agg2[0, :N], agg2[1, :N], g2, disc, b2p[None, :])

    return o[:, :N_CLASS]
